# baseline (speedup 1.0000x reference)
import functools

import jax
import jax.numpy as jnp
from jax.experimental import pallas as pl
from jax.experimental.pallas import tpu as pltpu


def _pool4_kernel(a_ref, b_ref, c_ref, d_ref,
                  oa_ref, ob_ref, oc_ref, od_ref, *, inv_hw):
    i = pl.program_id(1)
    k = pl.program_id(2)
    for x_ref, o_ref in ((a_ref, oa_ref), (b_ref, ob_ref),
                         (c_ref, oc_ref), (d_ref, od_ref)):
        x = x_ref[0].astype(jnp.float32)
        s = jnp.sum(x, axis=(0, 1)) * inv_hw
        row = s.astype(o_ref.dtype)[None, :]

        @pl.when(k == 0)
        def _():
            o_ref[pl.ds(i, 1), :] = row

        @pl.when(k != 0)
        def _():
            o_ref[pl.ds(i, 1), :] += row


def kernel(x_L_CC, x_L_MLO, x_R_CC, x_R_MLO):
    views = (x_L_CC, x_L_MLO, x_R_CC, x_R_MLO)
    n, c, h, w = views[0].shape
    dtype = views[0].dtype

    nhwc = [jnp.transpose(v, (0, 2, 3, 1)) for v in views]

    c_split = 2 if c % 256 == 0 else 1
    c_blk = c // c_split
    h_split = 2 if h % 2 == 0 else 1
    h_blk = h // h_split
    grid = (c_split, n, h_split)

    kernel_fn = functools.partial(_pool4_kernel, inv_hw=float(1.0 / (h * w)))

    in_spec = pl.BlockSpec((1, h_blk, w, c_blk), lambda j, i, k: (i, k, 0, j))
    out_spec = pl.BlockSpec((n, c_blk), lambda j, i, k: (0, j))
    itemsize = jnp.dtype(dtype).itemsize
    outs = pl.pallas_call(
        kernel_fn,
        out_shape=[jax.ShapeDtypeStruct((n, c), dtype)] * 4,
        grid=grid,
        in_specs=[in_spec] * 4,
        out_specs=[out_spec] * 4,
        compiler_params=pltpu.CompilerParams(
            dimension_semantics=("parallel", "arbitrary", "arbitrary"),
        ),
        cost_estimate=pl.CostEstimate(
            flops=4 * n * c * h * w,
            transcendentals=0,
            bytes_accessed=4 * (n * c * h * w + n * c) * itemsize,
        ),
    )(*nhwc)

    names = ("L-CC", "L-MLO", "R-CC", "R-MLO")
    return dict(zip(names, outs))


# speedup vs baseline: 1.5600x; 1.5600x over previous
import functools

import jax
import jax.numpy as jnp
from jax.experimental import pallas as pl
from jax.experimental.pallas import tpu as pltpu


def _pool4_kernel(a_ref, b_ref, c_ref, d_ref,
                  oa_ref, ob_ref, oc_ref, od_ref, *, inv_hw):
    i = pl.program_id(1)
    for x_ref, o_ref in ((a_ref, oa_ref), (b_ref, ob_ref),
                         (c_ref, oc_ref), (d_ref, od_ref)):
        x = x_ref[0].astype(jnp.float32)
        s = jnp.sum(x, axis=(0, 1)) * inv_hw
        o_ref[pl.ds(i, 1), :] = s.astype(o_ref.dtype)[None, :]


def kernel(x_L_CC, x_L_MLO, x_R_CC, x_R_MLO):
    views = (x_L_CC, x_L_MLO, x_R_CC, x_R_MLO)
    n, c, h, w = views[0].shape
    dtype = views[0].dtype

    nhwc = [jnp.transpose(v, (0, 2, 3, 1)) for v in views]

    c_split = 2 if c % 256 == 0 else 1
    c_blk = c // c_split
    grid = (c_split, n)

    kernel_fn = functools.partial(_pool4_kernel, inv_hw=float(1.0 / (h * w)))

    in_spec = pl.BlockSpec((1, h, w, c_blk), lambda j, i: (i, 0, 0, j))
    out_spec = pl.BlockSpec((n, c_blk), lambda j, i: (0, j))
    itemsize = jnp.dtype(dtype).itemsize
    outs = pl.pallas_call(
        kernel_fn,
        out_shape=[jax.ShapeDtypeStruct((n, c), dtype)] * 4,
        grid=grid,
        in_specs=[in_spec] * 4,
        out_specs=[out_spec] * 4,
        compiler_params=pltpu.CompilerParams(
            dimension_semantics=("parallel", "arbitrary"),
        ),
        cost_estimate=pl.CostEstimate(
            flops=4 * n * c * h * w,
            transcendentals=0,
            bytes_accessed=4 * (n * c * h * w + n * c) * itemsize,
        ),
    )(*nhwc)

    names = ("L-CC", "L-MLO", "R-CC", "R-MLO")
    return dict(zip(names, outs))
